# baseline (speedup 1.0000x reference)
import functools

import jax
import jax.numpy as jnp
from jax import lax
from jax.experimental import pallas as pl
from jax.experimental.pallas import tpu as pltpu
from jax.experimental.pallas import tpu_sc as plsc

_NC = 2
_NS = 16
_NW = _NC * _NS
_IDXW = 128


@functools.lru_cache(maxsize=None)
def _make_sc_gather(n, d, cr):
    rows_total = n // _IDXW
    rpw = rows_total // _NW
    chunks = rpw // cr
    assert chunks % 2 == 0 and chunks >= 6
    c = cr * _IDXW

    mesh = plsc.VectorSubcoreMesh(core_axis_name="c", subcore_axis_name="s")

    @functools.partial(
        pl.kernel,
        out_type=jax.ShapeDtypeStruct((n, d), jnp.float32),
        mesh=mesh,
        compiler_params=pltpu.CompilerParams(use_tc_tiling_on_sc=False),
        scratch_types=[
            pltpu.VMEM((2, c), jnp.int32),
            pltpu.VMEM((2, c, d), jnp.float32),
            pltpu.SemaphoreType.DMA,
            pltpu.SemaphoreType.DMA,
            pltpu.SemaphoreType.DMA,
            pltpu.SemaphoreType.DMA,
            pltpu.SemaphoreType.DMA,
            pltpu.SemaphoreType.DMA,
        ],
    )
    def gather_kernel(idx_hbm, table_hbm, out_hbm, idx_v, rows_v,
                      isem0, isem1, gsem0, gsem1, osem0, osem1):
        isem = (isem0, isem1)
        gsem = (gsem0, gsem1)
        osem = (osem0, osem1)
        wid = lax.axis_index("s") * _NC + lax.axis_index("c")
        row0 = wid * rpw

        def issue_idx(g, b):
            pltpu.async_copy(idx_hbm.at[pl.ds((row0 + g * cr) * _IDXW, c)],
                             idx_v.at[b], isem[b])

        def wait_idx(b):
            pltpu.make_async_copy(idx_hbm.at[pl.ds(0, c)],
                                  idx_v.at[b], isem[b]).wait()

        def issue_gathers(g, b):
            pltpu.async_copy(table_hbm.at[idx_v.at[b]], rows_v.at[b], gsem[b])

        def wait_gathers(b):
            pltpu.make_async_copy(out_hbm.at[pl.ds(0, c)],
                                  rows_v.at[b], gsem[b]).wait()

        def issue_store(g, b):
            pltpu.async_copy(rows_v.at[b],
                             out_hbm.at[pl.ds((row0 + g * cr) * _IDXW, c)],
                             osem[b])

        def wait_store(b):
            pltpu.make_async_copy(out_hbm.at[pl.ds(0, c)],
                                  rows_v.at[b], osem[b]).wait()

        def steady(g, b, first, last):
            bo = 1 - b
            wait_idx(b)
            if not first:
                wait_store(b)
            issue_gathers(g, b)
            wait_gathers(bo)
            issue_store(g - 1, bo)
            if not last:
                issue_idx(g + 1, bo)

        issue_idx(0, 0)
        issue_idx(1, 1)
        wait_idx(0)
        issue_gathers(0, 0)
        steady(1, 1, first=True, last=False)

        @pl.loop(0, (chunks - 4) // 2)
        def _pair(i):
            g0 = 2 + 2 * i
            steady(g0, 0, first=False, last=False)
            steady(g0 + 1, 1, first=False, last=False)

        steady(chunks - 2, 0, first=False, last=False)
        steady(chunks - 1, 1, first=False, last=True)
        wait_store(0)
        wait_gathers(1)
        issue_store(chunks - 1, 1)
        wait_store(1)

    return gather_kernel


def kernel(input_ids, weight):
    b, h = input_ids.shape
    n = b * h
    d = weight.shape[1]
    idx_flat = input_ids.reshape(n)
    out = _make_sc_gather(n, d, 4)(idx_flat, weight)
    return out.reshape(b, h, d)


# speedup vs baseline: 1.4929x; 1.4929x over previous
import functools

import jax
import jax.numpy as jnp
from jax import lax
from jax.experimental import pallas as pl
from jax.experimental.pallas import tpu as pltpu
from jax.experimental.pallas import tpu_sc as plsc

_NC = 2
_NS = 16
_NW = _NC * _NS


@functools.lru_cache(maxsize=None)
def _make_sc_gather(n, d, c):
    npw = n // _NW
    chunks = npw // c
    assert chunks % 2 == 0 and chunks >= 6

    mesh = plsc.VectorSubcoreMesh(core_axis_name="c", subcore_axis_name="s")

    @functools.partial(
        pl.kernel,
        out_type=jax.ShapeDtypeStruct((n, d), jnp.float32),
        mesh=mesh,
        scratch_types=[
            pltpu.VMEM((c,), jnp.int32),
            pltpu.VMEM((c,), jnp.int32),
            pltpu.VMEM((c, d), jnp.float32),
            pltpu.VMEM((c, d), jnp.float32),
            pltpu.SemaphoreType.DMA,
            pltpu.SemaphoreType.DMA,
            pltpu.SemaphoreType.DMA,
            pltpu.SemaphoreType.DMA,
            pltpu.SemaphoreType.DMA,
            pltpu.SemaphoreType.DMA,
        ],
    )
    def gather_kernel(idx_hbm, table_hbm, out_hbm, idx_v0, idx_v1,
                      rows_v0, rows_v1,
                      isem0, isem1, gsem0, gsem1, osem0, osem1):
        idx_v = (idx_v0, idx_v1)
        rows_v = (rows_v0, rows_v1)
        isem = (isem0, isem1)
        gsem = (gsem0, gsem1)
        osem = (osem0, osem1)
        wid = lax.axis_index("s") * _NC + lax.axis_index("c")
        i0 = wid * npw

        def issue_idx(g, b):
            pltpu.async_copy(idx_hbm.at[pl.ds(i0 + g * c, c)],
                             idx_v[b], isem[b])

        def wait_idx(b):
            pltpu.make_async_copy(idx_hbm.at[pl.ds(0, c)],
                                  idx_v[b], isem[b]).wait()

        def issue_gathers(g, b):
            @pl.loop(0, c, step=16)
            def _row(j):
                v = idx_v[b][pl.ds(j, 16)]
                for u in range(16):
                    pltpu.async_copy(table_hbm.at[v[u]],
                                     rows_v[b].at[j + u], gsem[b])

        def wait_gathers(b):
            pltpu.make_async_copy(out_hbm.at[pl.ds(0, c)],
                                  rows_v[b], gsem[b]).wait()

        def issue_store(g, b):
            pltpu.async_copy(rows_v[b],
                             out_hbm.at[pl.ds(i0 + g * c, c)],
                             osem[b])

        def wait_store(b):
            pltpu.make_async_copy(out_hbm.at[pl.ds(0, c)],
                                  rows_v[b], osem[b]).wait()

        def steady(g, b, first, last):
            bo = 1 - b
            wait_idx(b)
            if not first:
                wait_store(b)
            issue_gathers(g, b)
            wait_gathers(bo)
            issue_store(g - 1, bo)
            if not last:
                issue_idx(g + 1, bo)

        issue_idx(0, 0)
        issue_idx(1, 1)
        wait_idx(0)
        issue_gathers(0, 0)
        steady(1, 1, first=True, last=False)

        @pl.loop(0, (chunks - 4) // 2)
        def _pair(i):
            g0 = 2 + 2 * i
            steady(g0, 0, first=False, last=False)
            steady(g0 + 1, 1, first=False, last=False)

        steady(chunks - 2, 0, first=False, last=False)
        steady(chunks - 1, 1, first=False, last=True)
        wait_store(0)
        wait_gathers(1)
        issue_store(chunks - 1, 1)
        wait_store(1)

    return gather_kernel


def kernel(input_ids, weight):
    b, h = input_ids.shape
    n = b * h
    d = weight.shape[1]
    idx_flat = input_ids.reshape(n)
    out = _make_sc_gather(n, d, 256)(idx_flat, weight)
    return out.reshape(b, h, d)


# speedup vs baseline: 1.6103x; 1.0786x over previous
import functools

import jax
import jax.numpy as jnp
from jax import lax
from jax.experimental import pallas as pl
from jax.experimental.pallas import tpu as pltpu
from jax.experimental.pallas import tpu_sc as plsc

_NC = 2
_NS = 16
_NW = _NC * _NS


@functools.lru_cache(maxsize=None)
def _make_sc_gather(n, d, c):
    npw = n // _NW
    chunks = npw // c
    assert chunks % 2 == 0 and chunks >= 6

    mesh = plsc.VectorSubcoreMesh(core_axis_name="c", subcore_axis_name="s")

    @functools.partial(
        pl.kernel,
        out_type=jax.ShapeDtypeStruct((n, d), jnp.float32),
        mesh=mesh,
        scratch_types=[
            pltpu.VMEM((c,), jnp.int32),
            pltpu.VMEM((c,), jnp.int32),
            pltpu.VMEM((c, d), jnp.float32),
            pltpu.VMEM((c, d), jnp.float32),
            pltpu.SemaphoreType.DMA,
            pltpu.SemaphoreType.DMA,
            pltpu.SemaphoreType.DMA,
            pltpu.SemaphoreType.DMA,
            pltpu.SemaphoreType.DMA,
            pltpu.SemaphoreType.DMA,
        ],
    )
    def gather_kernel(idx_hbm, table_hbm, out_hbm, idx_v0, idx_v1,
                      rows_v0, rows_v1,
                      isem0, isem1, gsem0, gsem1, osem0, osem1):
        idx_v = (idx_v0, idx_v1)
        rows_v = (rows_v0, rows_v1)
        isem = (isem0, isem1)
        gsem = (gsem0, gsem1)
        osem = (osem0, osem1)
        wid = lax.axis_index("s") * _NC + lax.axis_index("c")
        i0 = wid * npw

        def issue_idx(g, b):
            pltpu.async_copy(idx_hbm.at[pl.ds(i0 + g * c, c)],
                             idx_v[b], isem[b])

        def wait_idx(b):
            pltpu.make_async_copy(idx_hbm.at[pl.ds(0, c)],
                                  idx_v[b], isem[b]).wait()

        def issue_gathers(g, b):
            @pl.loop(0, c, step=16)
            def _row(j):
                v = idx_v[b][pl.ds(j, 16)]
                for u in range(16):
                    r = v[u]
                    pltpu.async_copy(table_hbm.at[r // 8].at[r % 8],
                                     rows_v[b].at[j + u], gsem[b])

        def wait_gathers(b):
            pltpu.make_async_copy(out_hbm.at[pl.ds(0, c)],
                                  rows_v[b], gsem[b]).wait()

        def issue_store(g, b):
            pltpu.async_copy(rows_v[b],
                             out_hbm.at[pl.ds(i0 + g * c, c)],
                             osem[b])

        def wait_store(b):
            pltpu.make_async_copy(out_hbm.at[pl.ds(0, c)],
                                  rows_v[b], osem[b]).wait()

        def steady(g, b, first, last):
            bo = 1 - b
            wait_idx(b)
            if not first:
                wait_store(b)
            issue_gathers(g, b)
            wait_gathers(bo)
            issue_store(g - 1, bo)
            if not last:
                issue_idx(g + 1, bo)

        issue_idx(0, 0)
        issue_idx(1, 1)
        wait_idx(0)
        issue_gathers(0, 0)
        steady(1, 1, first=True, last=False)

        @pl.loop(0, (chunks - 4) // 2)
        def _pair(i):
            g0 = 2 + 2 * i
            steady(g0, 0, first=False, last=False)
            steady(g0 + 1, 1, first=False, last=False)

        steady(chunks - 2, 0, first=False, last=False)
        steady(chunks - 1, 1, first=False, last=True)
        wait_store(0)
        wait_gathers(1)
        issue_store(chunks - 1, 1)
        wait_store(1)

    return gather_kernel


def kernel(input_ids, weight):
    b, h = input_ids.shape
    n = b * h
    d = weight.shape[1]
    idx_flat = input_ids.reshape(n)
    w3d = weight.reshape(weight.shape[0] // 8, 8, d)
    out = _make_sc_gather(n, d, 256)(idx_flat, w3d)
    return out.reshape(b, h, d)


# speedup vs baseline: 1.7659x; 1.0967x over previous
import functools

import jax
import jax.numpy as jnp
from jax import lax
from jax.experimental import pallas as pl
from jax.experimental.pallas import tpu as pltpu
from jax.experimental.pallas import tpu_sc as plsc

_NC = 2
_NS = 16
_NW = _NC * _NS


@functools.lru_cache(maxsize=None)
def _make_sc_gather(n, d, c):
    npw = n // _NW
    chunks = npw // c
    assert chunks % 2 == 0 and chunks >= 6

    mesh = plsc.VectorSubcoreMesh(core_axis_name="c", subcore_axis_name="s")

    @functools.partial(
        pl.kernel,
        out_type=jax.ShapeDtypeStruct((n, d), jnp.float32),
        mesh=mesh,
        scratch_types=[
            pltpu.VMEM((c,), jnp.int32),
            pltpu.VMEM((c,), jnp.int32),
            pltpu.VMEM((c, d), jnp.float32),
            pltpu.VMEM((c, d), jnp.float32),
            pltpu.SemaphoreType.DMA,
            pltpu.SemaphoreType.DMA,
            pltpu.SemaphoreType.DMA,
            pltpu.SemaphoreType.DMA,
            pltpu.SemaphoreType.DMA,
            pltpu.SemaphoreType.DMA,
        ],
    )
    def gather_kernel(idx_hbm, table_hbm, out_hbm, idx_v0, idx_v1,
                      rows_v0, rows_v1,
                      isem0, isem1, gsem0, gsem1, osem0, osem1):
        idx_v = (idx_v0, idx_v1)
        rows_v = (rows_v0, rows_v1)
        isem = (isem0, isem1)
        gsem = (gsem0, gsem1)
        osem = (osem0, osem1)
        wid = lax.axis_index("s") * _NC + lax.axis_index("c")
        i0 = wid * npw

        def issue_idx(g, b):
            pltpu.async_copy(idx_hbm.at[pl.ds(i0 + g * c, c)],
                             idx_v[b], isem[b])

        def wait_idx(b):
            pltpu.make_async_copy(idx_hbm.at[pl.ds(0, c)],
                                  idx_v[b], isem[b]).wait()

        def issue_gathers(g, b):
            @pl.loop(0, c, step=16)
            def _row(j):
                v = idx_v[b][pl.ds(j, 16)]
                q = v >> 3
                s = v & 7
                for u in range(16):
                    pltpu.async_copy(table_hbm.at[q[u]].at[s[u]],
                                     rows_v[b].at[j + u], gsem[b])

        def wait_gathers(b):
            pltpu.make_async_copy(out_hbm.at[pl.ds(0, c)],
                                  rows_v[b], gsem[b]).wait()

        def issue_store(g, b):
            pltpu.async_copy(rows_v[b],
                             out_hbm.at[pl.ds(i0 + g * c, c)],
                             osem[b])

        def wait_store(b):
            pltpu.make_async_copy(out_hbm.at[pl.ds(0, c)],
                                  rows_v[b], osem[b]).wait()

        def steady(g, b, first, last):
            bo = 1 - b
            wait_idx(b)
            if not first:
                wait_store(b)
            issue_gathers(g, b)
            wait_gathers(bo)
            issue_store(g - 1, bo)
            if not last:
                issue_idx(g + 1, bo)

        issue_idx(0, 0)
        issue_idx(1, 1)
        wait_idx(0)
        issue_gathers(0, 0)
        steady(1, 1, first=True, last=False)

        @pl.loop(0, (chunks - 4) // 2)
        def _pair(i):
            g0 = 2 + 2 * i
            steady(g0, 0, first=False, last=False)
            steady(g0 + 1, 1, first=False, last=False)

        steady(chunks - 2, 0, first=False, last=False)
        steady(chunks - 1, 1, first=False, last=True)
        wait_store(0)
        wait_gathers(1)
        issue_store(chunks - 1, 1)
        wait_store(1)

    return gather_kernel


def kernel(input_ids, weight):
    b, h = input_ids.shape
    n = b * h
    d = weight.shape[1]
    idx_flat = input_ids.reshape(n)
    w3d = weight.reshape(weight.shape[0] // 8, 8, d)
    out = _make_sc_gather(n, d, 256)(idx_flat, w3d)
    return out.reshape(b, h, d)
